# Initial kernel scaffold; baseline (speedup 1.0000x reference)
import jax
import jax.numpy as jnp
from jax.experimental import pallas as pl


def kernel(prediction, instances, labels):
    raise NotImplementedError("write your pallas kernel here")


# speedup vs baseline: 8.7086x; 8.7086x over previous
import jax
import jax.numpy as jnp
from jax import lax
from jax.experimental import pallas as pl

H = W = 512
C = 32
F = 64
K = C * F
CH = 64
NCH = H // CH


def _loss_body(pred_ref, inst_ref, lab_ref, out_ref):
    b = pl.program_id(0)
    f32 = jnp.float32

    col = lax.broadcasted_iota(jnp.int32, (H, W), 1).astype(f32) * (1.0 / (W - 1))
    row = lax.broadcasted_iota(jnp.int32, (H, W), 0).astype(f32) * (1.0 / (H - 1))
    emb_x = jnp.tanh(pred_ref[0]) + col
    emb_y = jnp.tanh(pred_ref[1]) + row
    sgx = pred_ref[2]
    sgy = pred_ref[3]
    seed = 1.0 / (1.0 + jnp.exp(-pred_ref[4]))
    inst = inst_ref[...]
    lab = lab_ref[...]

    bg_seed = jnp.sum(jnp.where(lab == 0, seed * seed, 0.0))

    ac = lax.broadcasted_iota(jnp.int32, (CH, C, W), 1)
    af = lax.broadcasted_iota(jnp.int32, (CH, W, F), 2)
    m_fine = (lax.broadcasted_iota(jnp.int32, (F, F), 0)
              >= lax.broadcasted_iota(jnp.int32, (F, F), 1)).astype(f32)
    m_coarse = (lax.broadcasted_iota(jnp.int32, (C, C), 1)
                > lax.broadcasted_iota(jnp.int32, (C, C), 0)).astype(f32)
    ebar = ((lax.broadcasted_iota(jnp.int32, (C, F), 0) * F
             + lax.broadcasted_iota(jnp.int32, (C, F), 1)).astype(f32)
            + 0.5) * (2.0 / K)

    def body(iid, carry):
        inst_l, var_l, seed_l, obj = carry
        mb = inst == iid
        mf = mb.astype(f32)
        cnt = jnp.sum(mf)
        present = cnt > 0.0
        sc = jnp.where(present, cnt, 1.0)
        cx = jnp.sum(mf * emb_x) / sc
        cy = jnp.sum(mf * emb_y) / sc
        sx = jnp.sum(mf * sgx) / sc
        sy = jnp.sum(mf * sgy) / sc
        var_term = (jnp.sum(mf * (sgx - sx) ** 2)
                    + jnp.sum(mf * (sgy - sy) ** 2)) / (2.0 * sc)
        sex = jnp.exp(10.0 * sx)
        sey = jnp.exp(10.0 * sy)
        d = jnp.exp(-((emb_x - cx) ** 2 * sex + (emb_y - cy) ** 2 * sey))
        seed_term = jnp.sum(mf * (seed - d) ** 2)
        e = 1.0 - (d * 2.0 - 1.0) * (mf * 2.0 - 1.0)
        kbin = jnp.clip((e * (K / 2.0)).astype(jnp.int32), 0, K - 1)
        cc = jnp.right_shift(kbin, 6)
        ff = jnp.bitwise_and(kbin, F - 1)

        hist = jnp.zeros((2 * C, F), f32)
        for chunk in range(NCH):
            sl = slice(chunk * CH, (chunk + 1) * CH)
            c3 = cc[sl][:, None, :]
            m3 = mb[sl][:, None, :]
            eqc = c3 == ac
            u3 = jnp.concatenate(
                [jnp.where(eqc & m3, 1.0, 0.0),
                 jnp.where(eqc & (~m3), 1.0, 0.0)], axis=1).astype(jnp.bfloat16)
            v3 = (ff[sl][:, :, None] == af).astype(jnp.bfloat16)
            hc = lax.dot_general(u3, v3, (((2,), (1,)), ((0,), (0,))),
                                 preferred_element_type=f32)
            hist = hist + jnp.sum(hc, axis=0)

        hp = hist[:C]
        hn = hist[C:]
        sp = jnp.dot(hp, m_fine, preferred_element_type=f32) \
            + jnp.dot(m_coarse, jnp.sum(hp, axis=1, keepdims=True),
                      preferred_element_type=f32)
        sn = jnp.dot(hn, m_fine, preferred_element_type=f32) \
            + jnp.dot(m_coarse, jnp.sum(hn, axis=1, keepdims=True),
                      preferred_element_type=f32)
        p_tot = cnt

        def jac(p, n):
            return 1.0 - (p_tot - p) / jnp.maximum(p_tot + n, 1e-9)

        d_j = jac(sp, sn) - jac(sp - hp, sn - hn)
        lov = jnp.sum(ebar * d_j)

        z = jnp.where(present, 1.0, 0.0)
        return (inst_l + z * lov, var_l + z * var_term,
                seed_l + z * seed_term, obj + z)

    init = (jnp.float32(0), jnp.float32(0), jnp.float32(0), jnp.float32(0))
    inst_l, var_l, seed_fg, obj = lax.fori_loop(1, 16, body, init)

    has = obj > 0.0
    so = jnp.where(has, obj, 1.0)
    inst_l = jnp.where(has, inst_l / so, inst_l)
    var_l = jnp.where(has, var_l / so, var_l)
    seed_total = (bg_seed + seed_fg) / (H * W)
    loss_b = inst_l + 10.0 * var_l + seed_total

    prev = jnp.where(b == 0, jnp.zeros((1, 1), f32), out_ref[...])
    out_ref[...] = prev + 0.25 * loss_b


def kernel(prediction, instances, labels):
    loss = pl.pallas_call(
        _loss_body,
        grid=(4,),
        in_specs=[
            pl.BlockSpec((None, 5, H, W), lambda b: (b, 0, 0, 0)),
            pl.BlockSpec((None, H, W), lambda b: (b, 0, 0)),
            pl.BlockSpec((None, H, W), lambda b: (b, 0, 0)),
        ],
        out_specs=pl.BlockSpec((1, 1), lambda b: (0, 0)),
        out_shape=jax.ShapeDtypeStruct((1, 1), jnp.float32),
    )(prediction, instances, labels)
    return (loss[0, 0], jnp.zeros((), jnp.float32))


# speedup vs baseline: 35.0068x; 4.0198x over previous
import functools

import jax
import jax.numpy as jnp
from jax import lax
from jax.experimental import pallas as pl
from jax.experimental.pallas import tpu as pltpu
from jax.experimental.pallas import tpu_sc as plsc

H = W = 512
K = 2048
NKEY = 2 * K
NPIX = H * W
NINST = 15
NTASK = 4 * NINST

_info = plsc.get_sparse_core_info()
_NC, _NS, _L = _info.num_cores, _info.num_subcores, _info.num_lanes
_NW = _NC * _NS

SC_CHUNK = 16384
SC_NCHUNK = NPIX // SC_CHUNK


def _stage1_body(pred_ref, inst_ref, lab_ref, keys_ref, stats_ref):
    f32 = jnp.float32

    col = lax.broadcasted_iota(jnp.int32, (H, W), 1).astype(f32) * (1.0 / (W - 1))
    row = lax.broadcasted_iota(jnp.int32, (H, W), 0).astype(f32) * (1.0 / (H - 1))
    emb_x = jnp.tanh(pred_ref[0]) + col
    emb_y = jnp.tanh(pred_ref[1]) + row
    sgx = pred_ref[2]
    sgy = pred_ref[3]
    seed = 1.0 / (1.0 + jnp.exp(-pred_ref[4]))
    inst = inst_ref[...]
    lab = lab_ref[...]

    bg_seed = jnp.sum(jnp.where(lab == 0, seed * seed, 0.0))

    si = lax.broadcasted_iota(jnp.int32, (16, 128), 0)
    li = lax.broadcasted_iota(jnp.int32, (16, 128), 1)

    def body(iid, stats):
        mb = inst == iid
        mf = mb.astype(f32)
        cnt = jnp.sum(mf)
        present = cnt > 0.0
        sc = jnp.where(present, cnt, 1.0)
        cx = jnp.sum(mf * emb_x) / sc
        cy = jnp.sum(mf * emb_y) / sc
        sx = jnp.sum(mf * sgx) / sc
        sy = jnp.sum(mf * sgy) / sc
        var_term = (jnp.sum(mf * (sgx - sx) ** 2)
                    + jnp.sum(mf * (sgy - sy) ** 2)) / (2.0 * sc)
        sex = jnp.exp(10.0 * sx)
        sey = jnp.exp(10.0 * sy)
        d = jnp.exp(-((emb_x - cx) ** 2 * sex + (emb_y - cy) ** 2 * sey))
        seed_term = jnp.sum(mf * (seed - d) ** 2)
        e = 1.0 - (d * 2.0 - 1.0) * (mf * 2.0 - 1.0)
        kbin = jnp.clip((e * (K / 2.0)).astype(jnp.int32), 0, K - 1)
        key = kbin + jnp.where(mb, K, 0)
        keys_ref[pl.ds(iid - 1, 1)] = key[None]

        r = iid - 1
        stats = stats + jnp.where((si == r) & (li == 0), cnt, 0.0) \
            + jnp.where((si == r) & (li == 1), var_term, 0.0) \
            + jnp.where((si == r) & (li == 2), seed_term, 0.0)
        return stats

    stats = lax.fori_loop(1, 16, body, jnp.zeros((16, 128), f32))
    stats = stats + jnp.where((si == 0) & (li == 3), bg_seed, 0.0)
    stats_ref[...] = stats


def _hist_body(keys_hbm, out_hbm, chunk_v, hist_v, red_v, sem0, sem1):
    wid = lax.axis_index("s") * _NC + lax.axis_index("c")
    lane = lax.iota(jnp.int32, _L)
    loff = lane * NKEY
    ones = jnp.ones((_L,), jnp.float32)
    zeros16 = jnp.zeros((_L,), jnp.float32)
    sems = (sem0, sem1)

    for rnd in range((NTASK + _NW - 1) // _NW):
        task = wid + rnd * _NW

        @pl.when(task < NTASK)
        def _():
            def zbody(i, c):
                hist_v[pl.ds(i * _L, _L)] = zeros16
                return c
            lax.fori_loop(0, (NKEY * _L) // _L, zbody, 0)

            cps = {}
            cps[0] = pltpu.async_copy(
                keys_hbm.at[task, pl.ds(0, SC_CHUNK)], chunk_v.at[0], sems[0])
            for ch in range(SC_NCHUNK):
                buf = ch % 2
                if ch + 1 < SC_NCHUNK:
                    nbuf = (ch + 1) % 2
                    cps[ch + 1] = pltpu.async_copy(
                        keys_hbm.at[task, pl.ds((ch + 1) * SC_CHUNK, SC_CHUNK)],
                        chunk_v.at[nbuf], sems[nbuf])
                cps[ch].wait()

                def sbody(j, c):
                    base = j * (4 * _L)
                    for u in range(4):
                        kk = chunk_v[buf, pl.ds(base + u * _L, _L)]
                        plsc.addupdate_scatter(hist_v, [kk + loff], ones)
                    return c
                lax.fori_loop(0, SC_CHUNK // (4 * _L), sbody, 0)

            def rbody(j, c):
                acc = zeros16
                for ln in range(_L):
                    acc = acc + hist_v[pl.ds(ln * NKEY + j * _L, _L)]
                red_v[pl.ds(j * _L, _L)] = acc
                return c
            lax.fori_loop(0, NKEY // _L, rbody, 0)

            pltpu.sync_copy(red_v, out_hbm.at[task])


_hist_call = functools.partial(
    pl.kernel,
    mesh=plsc.VectorSubcoreMesh(core_axis_name="c", subcore_axis_name="s"),
    out_type=jax.ShapeDtypeStruct((NTASK, NKEY), jnp.float32),
    scratch_types=[
        pltpu.VMEM((2, SC_CHUNK), jnp.int32),
        pltpu.VMEM((NKEY * _L,), jnp.float32),
        pltpu.VMEM((NKEY,), jnp.float32),
        pltpu.SemaphoreType.DMA,
        pltpu.SemaphoreType.DMA,
    ],
    compiler_params=pltpu.CompilerParams(needs_layout_passes=False),
)(_hist_body)


def _stage3_body(hist_ref, stats_ref, out_ref):
    b = pl.program_id(0)
    f32 = jnp.float32

    hist = hist_ref[...]
    hn = hist[:, :K]
    hp = hist[:, K:]
    tri = (lax.broadcasted_iota(jnp.int32, (K, K), 0)
           >= lax.broadcasted_iota(jnp.int32, (K, K), 1)).astype(f32)
    sp = jnp.dot(hp, tri, preferred_element_type=f32)
    sn = jnp.dot(hn, tri, preferred_element_type=f32)

    stats = stats_ref[...]
    si = lax.broadcasted_iota(jnp.int32, (16, 128), 0)
    li = lax.broadcasted_iota(jnp.int32, (16, 128), 1)
    bg_seed = jnp.sum(jnp.where((si == 0) & (li == 3), stats, 0.0))
    cts = stats[0:NINST, 0:1]
    var_c = stats[0:NINST, 1:2]
    seed_c = stats[0:NINST, 2:3]

    present = cts > 0.0

    def jac(p, n):
        return 1.0 - (cts - p) / jnp.maximum(cts + n, 1e-9)

    d_j = jac(sp, sn) - jac(sp - hp, sn - hn)
    ebar = (lax.broadcasted_iota(jnp.int32, (1, K), 1).astype(f32)
            + 0.5) * (2.0 / K)
    lov = jnp.sum(ebar * d_j, axis=1, keepdims=True)

    z = jnp.where(present, 1.0, 0.0)
    inst_l = jnp.sum(z * lov)
    var_l = jnp.sum(z * var_c)
    seed_fg = jnp.sum(z * seed_c)
    obj = jnp.sum(z)

    has = obj > 0.0
    so = jnp.where(has, obj, 1.0)
    inst_l = jnp.where(has, inst_l / so, inst_l)
    var_l = jnp.where(has, var_l / so, var_l)
    seed_total = (bg_seed + seed_fg) / (H * W)
    loss_b = inst_l + 10.0 * var_l + seed_total

    prev = jnp.where(b == 0, jnp.zeros((1, 1), f32), out_ref[...])
    out_ref[...] = prev + 0.25 * loss_b


def kernel(prediction, instances, labels):
    keys, stats = pl.pallas_call(
        _stage1_body,
        grid=(4,),
        in_specs=[
            pl.BlockSpec((None, 5, H, W), lambda b: (b, 0, 0, 0)),
            pl.BlockSpec((None, H, W), lambda b: (b, 0, 0)),
            pl.BlockSpec((None, H, W), lambda b: (b, 0, 0)),
        ],
        out_specs=[
            pl.BlockSpec((None, NINST, H, W), lambda b: (b, 0, 0, 0)),
            pl.BlockSpec((None, 16, 128), lambda b: (b, 0, 0)),
        ],
        out_shape=[
            jax.ShapeDtypeStruct((4, NINST, H, W), jnp.int32),
            jax.ShapeDtypeStruct((4, 16, 128), jnp.float32),
        ],
    )(prediction, instances, labels)

    hist = _hist_call(keys.reshape(NTASK, NPIX))

    loss = pl.pallas_call(
        _stage3_body,
        grid=(4,),
        in_specs=[
            pl.BlockSpec((None, NINST, NKEY), lambda b: (b, 0, 0)),
            pl.BlockSpec((None, 16, 128), lambda b: (b, 0, 0)),
        ],
        out_specs=pl.BlockSpec((1, 1), lambda b: (0, 0)),
        out_shape=jax.ShapeDtypeStruct((1, 1), jnp.float32),
    )(hist.reshape(4, NINST, NKEY), stats)

    return (loss[0, 0], jnp.zeros((), jnp.float32))


# speedup vs baseline: 37.5818x; 1.0736x over previous
import functools

import jax
import jax.numpy as jnp
from jax import lax
from jax.experimental import pallas as pl
from jax.experimental.pallas import tpu as pltpu
from jax.experimental.pallas import tpu_sc as plsc

H = W = 512
K = 2048
NKEY = 2 * K
NPIX = H * W
NINST = 15
NTASK = 4 * NINST

_info = plsc.get_sparse_core_info()
_NC, _NS, _L = _info.num_cores, _info.num_subcores, _info.num_lanes
_NW = _NC * _NS

SC_CHUNK = 16384
SC_NCHUNK = NPIX // SC_CHUNK


def _stage1_body(pred_ref, inst_ref, lab_ref, keys_ref, stats_ref):
    f32 = jnp.float32

    col = lax.broadcasted_iota(jnp.int32, (H, W), 1).astype(f32) * (1.0 / (W - 1))
    row = lax.broadcasted_iota(jnp.int32, (H, W), 0).astype(f32) * (1.0 / (H - 1))
    emb_x = jnp.tanh(pred_ref[0]) + col
    emb_y = jnp.tanh(pred_ref[1]) + row
    sgx = pred_ref[2]
    sgy = pred_ref[3]
    seed = 1.0 / (1.0 + jnp.exp(-pred_ref[4]))
    inst = inst_ref[...]
    lab = lab_ref[...]

    bg_seed = jnp.sum(jnp.where(lab == 0, seed * seed, 0.0))

    si = lax.broadcasted_iota(jnp.int32, (16, 128), 0)
    li = lax.broadcasted_iota(jnp.int32, (16, 128), 1)

    def body(iid, stats):
        mb = inst == iid
        mf = mb.astype(f32)
        cnt = jnp.sum(mf)
        present = cnt > 0.0
        sc = jnp.where(present, cnt, 1.0)
        cx = jnp.sum(mf * emb_x) / sc
        cy = jnp.sum(mf * emb_y) / sc
        sx = jnp.sum(mf * sgx) / sc
        sy = jnp.sum(mf * sgy) / sc
        var_term = (jnp.sum(mf * (sgx - sx) ** 2)
                    + jnp.sum(mf * (sgy - sy) ** 2)) / (2.0 * sc)
        sex = jnp.exp(10.0 * sx)
        sey = jnp.exp(10.0 * sy)
        d = jnp.exp(-((emb_x - cx) ** 2 * sex + (emb_y - cy) ** 2 * sey))
        seed_term = jnp.sum(mf * (seed - d) ** 2)
        e = 1.0 - (d * 2.0 - 1.0) * (mf * 2.0 - 1.0)
        kbin = jnp.clip((e * (K / 2.0)).astype(jnp.int32), 0, K - 1)
        key = kbin + jnp.where(mb, K, 0)
        keys_ref[pl.ds(iid - 1, 1)] = key[None]

        r = iid - 1
        stats = stats + jnp.where((si == r) & (li == 0), cnt, 0.0) \
            + jnp.where((si == r) & (li == 1), var_term, 0.0) \
            + jnp.where((si == r) & (li == 2), seed_term, 0.0)
        return stats

    stats = lax.fori_loop(1, 16, body, jnp.zeros((16, 128), f32))
    stats = stats + jnp.where((si == 0) & (li == 3), bg_seed, 0.0)
    stats_ref[...] = stats


SC_UNROLL = 16


def _hist_body(keys_hbm, zeros_hbm, out_hbm, chunk_v, hist_v, red_v, sem0, sem1):
    wid = lax.axis_index("s") * _NC + lax.axis_index("c")
    lane = lax.iota(jnp.int32, _L)
    loff = lane * NKEY
    ones = jnp.ones((_L,), jnp.float32)
    zeros16 = jnp.zeros((_L,), jnp.float32)
    sems = (sem0, sem1)

    for rnd in range((NTASK + _NW - 1) // _NW):
        task = wid + rnd * _NW

        @pl.when(task < NTASK)
        def _():
            pltpu.sync_copy(zeros_hbm, hist_v)

            cps = {}
            cps[0] = pltpu.async_copy(
                keys_hbm.at[task, pl.ds(0, SC_CHUNK)], chunk_v.at[0], sems[0])
            for ch in range(SC_NCHUNK):
                buf = ch % 2
                if ch + 1 < SC_NCHUNK:
                    nbuf = (ch + 1) % 2
                    cps[ch + 1] = pltpu.async_copy(
                        keys_hbm.at[task, pl.ds((ch + 1) * SC_CHUNK, SC_CHUNK)],
                        chunk_v.at[nbuf], sems[nbuf])
                cps[ch].wait()

                def sbody(j, c):
                    base = j * (SC_UNROLL * _L)
                    for u in range(SC_UNROLL):
                        kk = chunk_v[buf, pl.ds(base + u * _L, _L)]
                        plsc.addupdate_scatter(hist_v, [kk + loff], ones)
                    return c
                lax.fori_loop(0, SC_CHUNK // (SC_UNROLL * _L), sbody, 0)

            def rbody(j, c):
                acc = zeros16
                for ln in range(_L):
                    acc = acc + hist_v[pl.ds(ln * NKEY + j * _L, _L)]
                red_v[pl.ds(j * _L, _L)] = acc
                return c
            lax.fori_loop(0, NKEY // _L, rbody, 0)

            pltpu.sync_copy(red_v, out_hbm.at[task])


_hist_call = functools.partial(
    pl.kernel,
    mesh=plsc.VectorSubcoreMesh(core_axis_name="c", subcore_axis_name="s"),
    out_type=jax.ShapeDtypeStruct((NTASK, NKEY), jnp.float32),
    scratch_types=[
        pltpu.VMEM((2, SC_CHUNK), jnp.int32),
        pltpu.VMEM((NKEY * _L,), jnp.float32),
        pltpu.VMEM((NKEY,), jnp.float32),
        pltpu.SemaphoreType.DMA,
        pltpu.SemaphoreType.DMA,
    ],
    compiler_params=pltpu.CompilerParams(needs_layout_passes=False),
)(_hist_body)


def _stage3_body(hist_ref, stats_ref, out_ref):
    b = pl.program_id(0)
    f32 = jnp.float32

    hist = hist_ref[...]
    hn = hist[:, :K]
    hp = hist[:, K:]
    tri = (lax.broadcasted_iota(jnp.int32, (K, K), 0)
           >= lax.broadcasted_iota(jnp.int32, (K, K), 1)).astype(f32)
    sp = jnp.dot(hp, tri, preferred_element_type=f32)
    sn = jnp.dot(hn, tri, preferred_element_type=f32)

    stats = stats_ref[...]
    si = lax.broadcasted_iota(jnp.int32, (16, 128), 0)
    li = lax.broadcasted_iota(jnp.int32, (16, 128), 1)
    bg_seed = jnp.sum(jnp.where((si == 0) & (li == 3), stats, 0.0))
    cts = stats[0:NINST, 0:1]
    var_c = stats[0:NINST, 1:2]
    seed_c = stats[0:NINST, 2:3]

    present = cts > 0.0

    def jac(p, n):
        return 1.0 - (cts - p) / jnp.maximum(cts + n, 1e-9)

    d_j = jac(sp, sn) - jac(sp - hp, sn - hn)
    ebar = (lax.broadcasted_iota(jnp.int32, (1, K), 1).astype(f32)
            + 0.5) * (2.0 / K)
    lov = jnp.sum(ebar * d_j, axis=1, keepdims=True)

    z = jnp.where(present, 1.0, 0.0)
    inst_l = jnp.sum(z * lov)
    var_l = jnp.sum(z * var_c)
    seed_fg = jnp.sum(z * seed_c)
    obj = jnp.sum(z)

    has = obj > 0.0
    so = jnp.where(has, obj, 1.0)
    inst_l = jnp.where(has, inst_l / so, inst_l)
    var_l = jnp.where(has, var_l / so, var_l)
    seed_total = (bg_seed + seed_fg) / (H * W)
    loss_b = inst_l + 10.0 * var_l + seed_total

    prev = jnp.where(b == 0, jnp.zeros((1, 1), f32), out_ref[...])
    out_ref[...] = prev + 0.25 * loss_b


def kernel(prediction, instances, labels):
    keys, stats = pl.pallas_call(
        _stage1_body,
        grid=(4,),
        in_specs=[
            pl.BlockSpec((None, 5, H, W), lambda b: (b, 0, 0, 0)),
            pl.BlockSpec((None, H, W), lambda b: (b, 0, 0)),
            pl.BlockSpec((None, H, W), lambda b: (b, 0, 0)),
        ],
        out_specs=[
            pl.BlockSpec((None, NINST, H, W), lambda b: (b, 0, 0, 0)),
            pl.BlockSpec((None, 16, 128), lambda b: (b, 0, 0)),
        ],
        out_shape=[
            jax.ShapeDtypeStruct((4, NINST, H, W), jnp.int32),
            jax.ShapeDtypeStruct((4, 16, 128), jnp.float32),
        ],
    )(prediction, instances, labels)

    hist = _hist_call(keys.reshape(NTASK, NPIX),
                      jnp.zeros((NKEY * _L,), jnp.float32))

    loss = pl.pallas_call(
        _stage3_body,
        grid=(4,),
        in_specs=[
            pl.BlockSpec((None, NINST, NKEY), lambda b: (b, 0, 0)),
            pl.BlockSpec((None, 16, 128), lambda b: (b, 0, 0)),
        ],
        out_specs=pl.BlockSpec((1, 1), lambda b: (0, 0)),
        out_shape=jax.ShapeDtypeStruct((1, 1), jnp.float32),
    )(hist.reshape(4, NINST, NKEY), stats)

    return (loss[0, 0], jnp.zeros((), jnp.float32))
